# baseline (speedup 1.0000x reference)
import functools

import jax
import jax.numpy as jnp
from jax import lax
from jax.experimental import pallas as pl
from jax.experimental.pallas import tpu as pltpu
from jax.experimental.pallas import tpu_sc as plsc

B = 128
T = 1024
C = 192
CP = 256
ROWS = B * T

NUM_CORES = 2
NUM_SUBCORES = 16
BATCHES_PER_WORKER = B // (NUM_CORES * NUM_SUBCORES)
BLK = 64
ITERS = BATCHES_PER_WORKER * T // BLK
CHUNKS_PER_BATCH = T // BLK


def _sc_spiral(x, abs_idx):
    mesh = plsc.VectorSubcoreMesh(core_axis_name="c", subcore_axis_name="s")

    @functools.partial(
        pl.kernel,
        mesh=mesh,
        compiler_params=pltpu.CompilerParams(use_tc_tiling_on_sc=True),
        out_type=(
            jax.ShapeDtypeStruct((B, T, C), jnp.float32),
            jax.ShapeDtypeStruct((ROWS, CP), jnp.float32),
        ),
        scratch_types=[
            pltpu.VMEM((2, BLK, C), jnp.float32),
            pltpu.VMEM((2, BLK, CP), jnp.float32),
            pltpu.VMEM((2, BLK), jnp.int32),
            pltpu.SemaphoreType.DMA((2,)),
            pltpu.SemaphoreType.DMA((2,)),
        ],
    )
    def k(x_hbm, idx_hbm, out_hbm, pad_hbm, nar_v, wide_v, idx_v, gsem, wsem):
        cid = lax.axis_index("c")
        sid = lax.axis_index("s")
        b0 = cid * (B // NUM_CORES) + sid * BATCHES_PER_WORKER

        def widen(p):
            def rows4(r4, carry):
                for dr in range(4):
                    r = r4 * 4 + dr
                    for c in range(C // 16):
                        wide_v[p, r, pl.ds(16 * c, 16)] = nar_v[
                            p, r, pl.ds(16 * c, 16)
                        ]
                return carry

            lax.fori_loop(0, BLK // 4, rows4, 0)

        def narrow(p):
            def rows4(r4, carry):
                for dr in range(4):
                    r = r4 * 4 + dr
                    for c in range(C // 16):
                        nar_v[p, r, pl.ds(16 * c, 16)] = wide_v[
                            p, r, pl.ds(16 * c, 16)
                        ]
                return carry

            lax.fori_loop(0, BLK // 4, rows4, 0)

        def bt(i):
            b = b0 + i // CHUNKS_PER_BATCH
            t0 = (i % CHUNKS_PER_BATCH) * BLK
            return b, t0

        def widen_pair(j, carry):
            rds = []
            for p in range(2):
                b, t0 = bt(j * 2 + p)
                rds.append(
                    pltpu.async_copy(
                        x_hbm.at[b].at[pl.ds(t0, BLK)], nar_v.at[p], gsem.at[p]
                    )
                )
            wds = []
            for p in range(2):
                b, t0 = bt(j * 2 + p)
                rds[p].wait()
                widen(p)
                wds.append(
                    pltpu.async_copy(
                        wide_v.at[p],
                        pad_hbm.at[pl.ds(b * T + t0, BLK)],
                        wsem.at[p],
                    )
                )
            for d in wds:
                d.wait()
            return carry

        lax.fori_loop(0, ITERS // 2, widen_pair, 0)
        plsc.subcore_barrier()

        def gather_pair(j, carry):
            ids, gds = [], []
            for p in range(2):
                b, t0 = bt(j * 2 + p)
                ids.append(
                    pltpu.async_copy(
                        idx_hbm.at[pl.ds(b * T + t0, BLK)],
                        idx_v.at[p],
                        gsem.at[p],
                    )
                )
            for p in range(2):
                ids[p].wait()
                gds.append(
                    pltpu.async_copy(
                        pad_hbm.at[idx_v.at[p]], wide_v.at[p], gsem.at[p]
                    )
                )
            ods = []
            for p in range(2):
                b, t0 = bt(j * 2 + p)
                gds[p].wait()
                narrow(p)
                ods.append(
                    pltpu.async_copy(
                        nar_v.at[p],
                        out_hbm.at[b].at[pl.ds(t0, BLK)],
                        wsem.at[p],
                    )
                )
            for d in ods:
                d.wait()
            return carry

        lax.fori_loop(0, ITERS // 2, gather_pair, 0)

    out, _ = k(x, abs_idx)
    return out


def kernel(x, forward_shuffle_idx):
    idx = forward_shuffle_idx.astype(jnp.int32)
    abs_idx = (
        jnp.arange(B, dtype=jnp.int32)[:, None] * T + idx[None, :]
    ).reshape(ROWS)
    return _sc_spiral(x, abs_idx)


# speedup vs baseline: 1.1182x; 1.1182x over previous
import functools

import jax
import jax.numpy as jnp
from jax import lax
from jax.experimental import pallas as pl
from jax.experimental.pallas import tpu as pltpu
from jax.experimental.pallas import tpu_sc as plsc

B = 128
T = 1024
C = 192
CP = 256
ROWS = B * T

NUM_CORES = 2
NUM_SUBCORES = 16
BATCHES_PER_WORKER = B // (NUM_CORES * NUM_SUBCORES)
BLK = 128
ITERS = BATCHES_PER_WORKER * T // BLK
CHUNKS_PER_BATCH = T // BLK


def _sc_spiral(x, abs_idx):
    mesh = plsc.VectorSubcoreMesh(core_axis_name="c", subcore_axis_name="s")

    @functools.partial(
        pl.kernel,
        mesh=mesh,
        compiler_params=pltpu.CompilerParams(use_tc_tiling_on_sc=True),
        out_type=(
            jax.ShapeDtypeStruct((B, T, C), jnp.float32),
            jax.ShapeDtypeStruct((ROWS, CP), jnp.float32),
        ),
        scratch_types=[
            pltpu.VMEM((BLK, C), jnp.float32),
            pltpu.VMEM((BLK, CP), jnp.float32),
            pltpu.VMEM((BLK,), jnp.int32),
            pltpu.SemaphoreType.DMA,
            pltpu.SemaphoreType.DMA,
        ],
    )
    def k(x_hbm, idx_hbm, out_hbm, pad_hbm, nar_v, wide_v, idx_v, gsem, wsem):
        cid = lax.axis_index("c")
        sid = lax.axis_index("s")
        b0 = cid * (B // NUM_CORES) + sid * BATCHES_PER_WORKER

        def widen_rows(r, carry):
            for c in range(C // 16):
                wide_v[r, pl.ds(16 * c, 16)] = nar_v[r, pl.ds(16 * c, 16)]
            return carry

        def narrow_rows(r, carry):
            for c in range(C // 16):
                nar_v[r, pl.ds(16 * c, 16)] = wide_v[r, pl.ds(16 * c, 16)]
            return carry

        def widen_block(i, carry):
            b = b0 + i // CHUNKS_PER_BATCH
            t0 = (i % CHUNKS_PER_BATCH) * BLK
            pltpu.async_copy(
                x_hbm.at[b].at[pl.ds(t0, BLK)], nar_v, gsem
            ).wait()
            lax.fori_loop(0, BLK, widen_rows, 0)
            pltpu.async_copy(
                wide_v, pad_hbm.at[pl.ds(b * T + t0, BLK)], wsem
            ).wait()
            return carry

        lax.fori_loop(0, ITERS, widen_block, 0)
        plsc.subcore_barrier()

        def gather_chunk(i, carry):
            b = b0 + i // CHUNKS_PER_BATCH
            t0 = (i % CHUNKS_PER_BATCH) * BLK
            pltpu.async_copy(
                idx_hbm.at[pl.ds(b * T + t0, BLK)], idx_v, gsem
            ).wait()
            pltpu.async_copy(pad_hbm.at[idx_v], wide_v, gsem).wait()
            lax.fori_loop(0, BLK, narrow_rows, 0)
            pltpu.async_copy(
                nar_v, out_hbm.at[b].at[pl.ds(t0, BLK)], wsem
            ).wait()
            return carry

        lax.fori_loop(0, ITERS, gather_chunk, 0)

    out, _ = k(x, abs_idx)
    return out


def kernel(x, forward_shuffle_idx):
    idx = forward_shuffle_idx.astype(jnp.int32)
    abs_idx = (
        jnp.arange(B, dtype=jnp.int32)[:, None] * T + idx[None, :]
    ).reshape(ROWS)
    return _sc_spiral(x, abs_idx)
